# Initial kernel scaffold; baseline (speedup 1.0000x reference)
import jax
import jax.numpy as jnp
from jax.experimental import pallas as pl


def kernel(x, edge_index, edge_norm, edge_type, seq_lengths, umask, nodal_attn, avec, comp, basis, root, bias1, w_rel, w_root, bias2, w_lin, b_lin, w_fc, b_fc):
    raise NotImplementedError("write your pallas kernel here")


# speedup vs baseline: 4.0091x; 4.0091x over previous
import functools

import jax
import jax.numpy as jnp
from jax import lax
from jax.experimental import pallas as pl
from jax.experimental.pallas import tpu as pltpu
from jax.experimental.pallas import tpu_sc as plsc

N = 10000
E = 320000
D = 128
H = 64
R = 16
RN = R * N

NC = 2
NS = 16
L = 16

PB = 128
NB = 79
ET = NB * PB
EPAD = NC * NS * ET
EHALF = NS * ET
NBA = 2 * NB

ACC_ROWS = 10112
ROWS_PT = ACC_ROWS // NS
CNT_PT = 10640
CNTSZ = NS * CNT_PT

_i32 = jnp.int32
_f32 = jnp.float32


def _w_body(comp_ref, basis_ref, w_ref):
    w_ref[...] = jnp.dot(comp_ref[...], basis_ref[...],
                         preferred_element_type=_f32)


def _xw_body(x_ref, w_ref, xw_ref):
    xw_ref[...] = jnp.dot(x_ref[...], w_ref[0],
                          preferred_element_type=_f32)[None]


def _out1_body(x_ref, p0_ref, p1_ref, root_ref, b1_ref, o_ref):
    o_ref[...] = (p0_ref[...] + p1_ref[...] + b1_ref[...]
                  + jnp.dot(x_ref[...], root_ref[...],
                            preferred_element_type=_f32))


def _head_body(x_ref, o1_ref, a0_ref, a1_ref, wrel_ref, wroot_ref, b2_ref,
               wlx_ref, wlh_ref, bl_ref, wfc_ref, bfc_ref, o_ref):
    agg2 = a0_ref[...] + a1_ref[...]
    out2 = (jnp.dot(agg2, wrel_ref[...], preferred_element_type=_f32)
            + jnp.dot(o1_ref[...], wroot_ref[...], preferred_element_type=_f32)
            + b2_ref[...])
    hidden = (jnp.dot(x_ref[...], wlx_ref[...], preferred_element_type=_f32)
              + jnp.dot(out2, wlh_ref[...], preferred_element_type=_f32)
              + bl_ref[...])
    hidden = jnp.maximum(hidden, 0.0)
    logits = jnp.dot(hidden, wfc_ref[...], preferred_element_type=_f32) \
        + bfc_ref[...]
    m = jnp.max(logits, axis=1, keepdims=True)
    s = logits - m
    lse = jnp.log(jnp.sum(jnp.exp(s), axis=1, keepdims=True))
    o_ref[...] = s - lse


def _zero_acc_slice(rows, acc_sh, s):
    def _fill_rows(i, _):
        for k in range(H // L):
            rows[i, pl.ds(k * L, L)] = jnp.zeros((L,), _f32)
        return 0
    lax.fori_loop(0, PB, _fill_rows, 0)
    for q in range(ROWS_PT // PB):
        pltpu.sync_copy(rows, acc_sh.at[pl.ds(s * ROWS_PT + q * PB, PB)])
    rem = ROWS_PT % PB
    if rem:
        pltpu.sync_copy(
            rows.at[pl.ds(0, rem)],
            acc_sh.at[pl.ds(s * ROWS_PT + (ROWS_PT // PB) * PB, rem)])


def _sc1_body(et_hbm, dst_hbm, src_hbm, xw_hbm, out_hbm,
              ebuf, idx_c, idx_g, sidx, ones, sval, zc, rows, gsem,
              cnt_sh, acc_sh):
    c = lax.axis_index("c")
    s = lax.axis_index("s")

    lanes = lax.broadcasted_iota(_i32, (L,), 0)

    def _fill_zc(i, _):
        zc[pl.ds(i * L, L)] = jnp.zeros((L,), _f32)
        return 0
    lax.fori_loop(0, CNT_PT // L, _fill_zc, 0)

    ones[...] = jnp.ones((L,), _f32)

    pltpu.sync_copy(zc, cnt_sh.at[pl.ds(s * CNT_PT, CNT_PT)])
    _zero_acc_slice(rows, acc_sh, s)
    plsc.subcore_barrier()

    def _stage_a(st, _):
        abase = s * (2 * ET) + st * ET
        pltpu.sync_copy(et_hbm.at[pl.ds(abase, ET)], ebuf.at[pl.ds(0, ET)])
        pltpu.sync_copy(dst_hbm.at[pl.ds(abase, ET)], ebuf.at[pl.ds(ET, ET)])

        def _cidx(g, _):
            t16 = ebuf[pl.ds(g * L, L)]
            d16 = ebuf[pl.ds(ET + g * L, L)]
            idx_c[pl.ds(g * L, L)] = t16 * N + d16
            return 0
        lax.fori_loop(0, ET // L, _cidx, 0)

        def _scat(b, _):
            for q in range(PB // L):
                sidx[...] = idx_c[pl.ds(b * PB + q * L, L)]
                pltpu.sync_copy(ones, cnt_sh.at[sidx], add=True)
            return 0
        lax.fori_loop(0, NB, _scat, 0)
        return 0
    lax.fori_loop(0, 2, _stage_a, 0)
    plsc.subcore_barrier()

    pltpu.sync_copy(cnt_sh.at[pl.ds(s * CNT_PT, CNT_PT)], zc)

    def _recip(i, _):
        v = zc[pl.ds(i * L, L)]
        zc[pl.ds(i * L, L)] = 1.0 / jnp.maximum(v, 1.0)
        return 0
    lax.fori_loop(0, CNT_PT // L, _recip, 0)
    pltpu.sync_copy(zc, cnt_sh.at[pl.ds(s * CNT_PT, CNT_PT)])
    plsc.subcore_barrier()

    base = c * EHALF + s * ET
    pltpu.sync_copy(et_hbm.at[pl.ds(base, ET)], ebuf.at[pl.ds(0, ET)])
    pltpu.sync_copy(dst_hbm.at[pl.ds(base, ET)], ebuf.at[pl.ds(ET, ET)])
    pltpu.sync_copy(src_hbm.at[pl.ds(base, ET)], ebuf.at[pl.ds(2 * ET, ET)])

    def _bidx(g, _):
        t16 = ebuf[pl.ds(g * L, L)]
        d16 = ebuf[pl.ds(ET + g * L, L)]
        s16 = ebuf[pl.ds(2 * ET + g * L, L)]
        idx_c[pl.ds(g * L, L)] = t16 * N + d16
        idx_g[pl.ds(g * L, L)] = jnp.minimum(t16 * N + s16, RN - 1)
        return 0
    lax.fori_loop(0, ET // L, _bidx, 0)

    fidx = [lanes + f * L for f in range(H // L)]

    def _batch(b, _):
        pltpu.async_copy(xw_hbm.at[idx_g.at[pl.ds(b * PB, PB)]], rows,
                         gsem).wait()
        pltpu.sync_copy(cnt_sh.at[idx_c.at[pl.ds(b * PB, PB)]], sval)

        def _grp(j, _):
            for k in range(L):
                e = j * L + k
                eidx = jnp.full((L,), e, _i32)
                spl = plsc.load_gather(sval, [eidx])
                for f in range(H // L):
                    v = plsc.load_gather(rows, [eidx, fidx[f]])
                    plsc.store_scatter(rows, [eidx, fidx[f]], v * spl)
            return 0
        lax.fori_loop(0, PB // L, _grp, 0)

        for q in range(PB // L):
            sidx[...] = ebuf[pl.ds(ET + b * PB + q * L, L)]
            pltpu.sync_copy(rows.at[pl.ds(q * L, L)],
                            acc_sh.at[sidx], add=True)
        return 0
    lax.fori_loop(0, NB, _batch, 0)
    plsc.subcore_barrier()

    pltpu.sync_copy(acc_sh.at[pl.ds(s * ROWS_PT, ROWS_PT)],
                    out_hbm.at[pl.ds(c * ACC_ROWS + s * ROWS_PT, ROWS_PT)])


def _sc2_body(src_hbm, dst_hbm, o1_hbm, out_hbm,
              ebuf, sidx, rows, gsem, acc_sh):
    c = lax.axis_index("c")
    s = lax.axis_index("s")

    _zero_acc_slice(rows, acc_sh, s)
    plsc.subcore_barrier()

    base = c * EHALF + s * ET
    pltpu.sync_copy(src_hbm.at[pl.ds(base, ET)], ebuf.at[pl.ds(0, ET)])
    pltpu.sync_copy(dst_hbm.at[pl.ds(base, ET)], ebuf.at[pl.ds(ET, ET)])

    def _batch(b, _):
        pltpu.async_copy(o1_hbm.at[ebuf.at[pl.ds(b * PB, PB)]], rows,
                         gsem).wait()
        for q in range(PB // L):
            sidx[...] = ebuf[pl.ds(ET + b * PB + q * L, L)]
            pltpu.sync_copy(rows.at[pl.ds(q * L, L)],
                            acc_sh.at[sidx], add=True)
        return 0
    lax.fori_loop(0, NB, _batch, 0)
    plsc.subcore_barrier()

    pltpu.sync_copy(acc_sh.at[pl.ds(s * ROWS_PT, ROWS_PT)],
                    out_hbm.at[pl.ds(c * ACC_ROWS + s * ROWS_PT, ROWS_PT)])


_MESH = plsc.VectorSubcoreMesh(core_axis_name="c", subcore_axis_name="s",
                               num_cores=NC, num_subcores=NS)

_SC_PARAMS = pltpu.CompilerParams(needs_layout_passes=False,
                                  use_tc_tiling_on_sc=False)

_sc1 = functools.partial(
    pl.kernel, _sc1_body,
    out_type=jax.ShapeDtypeStruct((NC * ACC_ROWS, H), _f32),
    mesh=_MESH,
    compiler_params=_SC_PARAMS,
    scratch_types=[
        pltpu.VMEM((3 * ET,), _i32),
        pltpu.VMEM((ET,), _i32),
        pltpu.VMEM((ET,), _i32),
        pltpu.VMEM((L,), _i32),
        pltpu.VMEM((L,), _f32),
        pltpu.VMEM((PB,), _f32),
        pltpu.VMEM((CNT_PT,), _f32),
        pltpu.VMEM((PB, H), _f32),
        pltpu.SemaphoreType.DMA,
        pltpu.VMEM_SHARED((CNTSZ,), _f32),
        pltpu.VMEM_SHARED((ACC_ROWS, H), _f32),
    ],
)

_sc2 = functools.partial(
    pl.kernel, _sc2_body,
    out_type=jax.ShapeDtypeStruct((NC * ACC_ROWS, H), _f32),
    mesh=_MESH,
    compiler_params=_SC_PARAMS,
    scratch_types=[
        pltpu.VMEM((2 * ET,), _i32),
        pltpu.VMEM((L,), _i32),
        pltpu.VMEM((PB, H), _f32),
        pltpu.SemaphoreType.DMA,
        pltpu.VMEM_SHARED((ACC_ROWS, H), _f32),
    ],
)


def kernel(x, edge_index, edge_norm, edge_type, seq_lengths, umask,
           nodal_attn, avec, comp, basis, root, bias1, w_rel, w_root,
           bias2, w_lin, b_lin, w_fc, b_fc):
    del edge_norm, seq_lengths, umask, nodal_attn, avec

    src = edge_index[0].astype(_i32)
    dst = edge_index[1].astype(_i32)
    et = edge_type.astype(_i32)
    pad = EPAD - E
    srcp = jnp.concatenate([src, jnp.zeros((pad,), _i32)])
    dstp = jnp.concatenate([dst, jnp.full((pad,), N, _i32)])
    etp = jnp.concatenate([et, jnp.full((pad,), R, _i32)])

    basis2 = basis.reshape(basis.shape[0], D * H)
    w_flat = pl.pallas_call(
        _w_body,
        out_shape=jax.ShapeDtypeStruct((R, D * H), _f32),
    )(comp, basis2)
    w_all = w_flat.reshape(R, D, H)

    xw = pl.pallas_call(
        _xw_body,
        grid=(R,),
        in_specs=[
            pl.BlockSpec((N, D), lambda r: (0, 0)),
            pl.BlockSpec((1, D, H), lambda r: (r, 0, 0)),
        ],
        out_specs=pl.BlockSpec((1, N, H), lambda r: (r, 0, 0)),
        out_shape=jax.ShapeDtypeStruct((R, N, H), _f32),
    )(x, w_all)
    xw2d = xw.reshape(RN, H)

    relpart = _sc1()(etp, dstp, srcp, xw2d)

    out1 = pl.pallas_call(
        _out1_body,
        out_shape=jax.ShapeDtypeStruct((N, H), _f32),
    )(x, relpart[:N], relpart[ACC_ROWS:ACC_ROWS + N], root, bias1[None, :])

    aggpart = _sc2()(srcp, dstp, out1)

    log_prob = pl.pallas_call(
        _head_body,
        out_shape=jax.ShapeDtypeStruct((N, b_fc.shape[0]), _f32),
    )(x, out1, aggpart[:N], aggpart[ACC_ROWS:ACC_ROWS + N],
      w_rel, w_root, bias2[None, :], w_lin[:D], w_lin[D:],
      b_lin[None, :], w_fc, b_fc[None, :])
    return log_prob


# speedup vs baseline: 6.6081x; 1.6483x over previous
import functools

import jax
import jax.numpy as jnp
from jax import lax
from jax.experimental import pallas as pl
from jax.experimental.pallas import tpu as pltpu
from jax.experimental.pallas import tpu_sc as plsc

N = 10000
E = 320000
D = 128
H = 64
R = 16
RN = R * N

NC = 2
NS = 16
L = 16

PB = 128
NB = 79
ET = NB * PB
EPAD = NC * NS * ET
EHALF = NS * ET
NBA = 2 * NB

ACC_ROWS = 10112
ROWS_PT = ACC_ROWS // NS
CNT_PT = 10640
CNTSZ = NS * CNT_PT

_i32 = jnp.int32
_f32 = jnp.float32


def _w_body(comp_ref, basis_ref, w_ref):
    w_ref[...] = jnp.dot(comp_ref[...], basis_ref[...],
                         preferred_element_type=_f32)


def _xw_body(x_ref, w_ref, xw_ref):
    xw_ref[...] = jnp.dot(x_ref[...], w_ref[0],
                          preferred_element_type=_f32)[None]


def _out1_body(x_ref, p0_ref, p1_ref, root_ref, b1_ref, o_ref):
    o_ref[...] = (p0_ref[...] + p1_ref[...] + b1_ref[...]
                  + jnp.dot(x_ref[...], root_ref[...],
                            preferred_element_type=_f32))


def _head_body(x_ref, o1_ref, a0_ref, a1_ref, wrel_ref, wroot_ref, b2_ref,
               wlx_ref, wlh_ref, bl_ref, wfc_ref, bfc_ref, o_ref):
    agg2 = a0_ref[...] + a1_ref[...]
    out2 = (jnp.dot(agg2, wrel_ref[...], preferred_element_type=_f32)
            + jnp.dot(o1_ref[...], wroot_ref[...], preferred_element_type=_f32)
            + b2_ref[...])
    hidden = (jnp.dot(x_ref[...], wlx_ref[...], preferred_element_type=_f32)
              + jnp.dot(out2, wlh_ref[...], preferred_element_type=_f32)
              + bl_ref[...])
    hidden = jnp.maximum(hidden, 0.0)
    logits = jnp.dot(hidden, wfc_ref[...], preferred_element_type=_f32) \
        + bfc_ref[...]
    m = jnp.max(logits, axis=1, keepdims=True)
    s = logits - m
    lse = jnp.log(jnp.sum(jnp.exp(s), axis=1, keepdims=True))
    o_ref[...] = s - lse


def _zero_acc_slice(rows, acc_sh, s):
    def _fill_rows(i, _):
        for k in range(H // L):
            rows[0, i, pl.ds(k * L, L)] = jnp.zeros((L,), _f32)
        return 0
    lax.fori_loop(0, PB, _fill_rows, 0)
    for q in range(ROWS_PT // PB):
        pltpu.sync_copy(rows.at[0],
                        acc_sh.at[pl.ds(s * ROWS_PT + q * PB, PB)])
    rem = ROWS_PT % PB
    if rem:
        pltpu.sync_copy(
            rows.at[0, pl.ds(0, rem)],
            acc_sh.at[pl.ds(s * ROWS_PT + (ROWS_PT // PB) * PB, rem)])


def _sc1_body(et_hbm, dst_hbm, src_hbm, xw_hbm, out_hbm,
              ebuf, idx_c, idx_g, sidx, ones, sval, zc, rows,
              gsem, ssem, wsem, cnt_sh, acc_sh):
    c = lax.axis_index("c")
    s = lax.axis_index("s")

    lanes = lax.broadcasted_iota(_i32, (L,), 0)

    def _fill_zc(i, _):
        zc[pl.ds(i * L, L)] = jnp.zeros((L,), _f32)
        return 0
    lax.fori_loop(0, CNT_PT // L, _fill_zc, 0)

    ones[...] = jnp.ones((L,), _f32)

    pltpu.sync_copy(zc, cnt_sh.at[pl.ds(s * CNT_PT, CNT_PT)])
    _zero_acc_slice(rows, acc_sh, s)
    plsc.subcore_barrier()

    def _stage_a(st, _):
        abase = s * (2 * ET) + st * ET
        pltpu.sync_copy(et_hbm.at[pl.ds(abase, ET)], ebuf.at[pl.ds(0, ET)])
        pltpu.sync_copy(dst_hbm.at[pl.ds(abase, ET)], ebuf.at[pl.ds(ET, ET)])

        def _cidx(g, _):
            t16 = ebuf[pl.ds(g * L, L)]
            d16 = ebuf[pl.ds(ET + g * L, L)]
            idx_c[pl.ds(g * L, L)] = t16 * N + d16
            return 0
        lax.fori_loop(0, ET // L, _cidx, 0)

        def _scat(b, _):
            for q in range(PB // L):
                sidx[q, pl.ds(0, L)] = idx_c[pl.ds(b * PB + q * L, L)]
                pltpu.async_copy(ones, cnt_sh.at[sidx.at[q]], wsem,
                                 add=True)
            for q in range(PB // L):
                pltpu.make_async_copy(ones, cnt_sh.at[sidx.at[q]],
                                      wsem).wait()
            return 0
        lax.fori_loop(0, NB, _scat, 0)
        return 0
    lax.fori_loop(0, 2, _stage_a, 0)
    plsc.subcore_barrier()

    pltpu.sync_copy(cnt_sh.at[pl.ds(s * CNT_PT, CNT_PT)], zc)

    def _recip(i, _):
        v = zc[pl.ds(i * L, L)]
        zc[pl.ds(i * L, L)] = 1.0 / jnp.maximum(v, 1.0)
        return 0
    lax.fori_loop(0, CNT_PT // L, _recip, 0)
    pltpu.sync_copy(zc, cnt_sh.at[pl.ds(s * CNT_PT, CNT_PT)])
    plsc.subcore_barrier()

    base = c * EHALF + s * ET
    pltpu.sync_copy(et_hbm.at[pl.ds(base, ET)], ebuf.at[pl.ds(0, ET)])
    pltpu.sync_copy(dst_hbm.at[pl.ds(base, ET)], ebuf.at[pl.ds(ET, ET)])
    pltpu.sync_copy(src_hbm.at[pl.ds(base, ET)], ebuf.at[pl.ds(2 * ET, ET)])

    def _bidx(g, _):
        t16 = ebuf[pl.ds(g * L, L)]
        d16 = ebuf[pl.ds(ET + g * L, L)]
        s16 = ebuf[pl.ds(2 * ET + g * L, L)]
        idx_c[pl.ds(g * L, L)] = t16 * N + d16
        idx_g[pl.ds(g * L, L)] = jnp.minimum(t16 * N + s16, RN - 1)
        return 0
    lax.fori_loop(0, ET // L, _bidx, 0)

    fidx = [lanes + f * L for f in range(H // L)]

    def _fire(b, p):
        pltpu.async_copy(xw_hbm.at[idx_g.at[pl.ds(b * PB, PB)]],
                         rows.at[p], gsem.at[p])
        pltpu.async_copy(cnt_sh.at[idx_c.at[pl.ds(b * PB, PB)]],
                         sval.at[p], ssem.at[p])

    _fire(0, 0)

    def _batch(b, _):
        p = lax.rem(b, 2)
        pn = 1 - p

        @pl.when(b + 1 < NB)
        def _():
            _fire(b + 1, pn)

        pltpu.make_async_copy(xw_hbm.at[idx_g.at[pl.ds(b * PB, PB)]],
                              rows.at[p], gsem.at[p]).wait()
        pltpu.make_async_copy(cnt_sh.at[idx_c.at[pl.ds(b * PB, PB)]],
                              sval.at[p], ssem.at[p]).wait()
        pidx = jnp.full((L,), p, _i32)

        def _grp(j, _):
            for k in range(L):
                e = j * L + k
                eidx = jnp.full((L,), e, _i32)
                spl = plsc.load_gather(sval, [pidx, eidx])
                for f in range(H // L):
                    v = plsc.load_gather(rows, [pidx, eidx, fidx[f]])
                    plsc.store_scatter(rows, [pidx, eidx, fidx[f]],
                                       v * spl)
            return 0
        lax.fori_loop(0, PB // L, _grp, 0)

        for q in range(PB // L):
            sidx[q, pl.ds(0, L)] = ebuf[pl.ds(ET + b * PB + q * L, L)]
            pltpu.async_copy(rows.at[p, pl.ds(q * L, L)],
                             acc_sh.at[sidx.at[q]], wsem, add=True)
        for q in range(PB // L):
            pltpu.make_async_copy(rows.at[p, pl.ds(q * L, L)],
                                  acc_sh.at[sidx.at[q]], wsem).wait()
        return 0
    lax.fori_loop(0, NB, _batch, 0)
    plsc.subcore_barrier()

    pltpu.sync_copy(acc_sh.at[pl.ds(s * ROWS_PT, ROWS_PT)],
                    out_hbm.at[pl.ds(c * ACC_ROWS + s * ROWS_PT, ROWS_PT)])


def _sc2_body(src_hbm, dst_hbm, o1_hbm, out_hbm,
              ebuf, sidx, rows, gsem, wsem, acc_sh, o1_sh):
    c = lax.axis_index("c")
    s = lax.axis_index("s")

    lastn = N - (NS - 1) * ROWS_PT

    @pl.when(s < NS - 1)
    def _():
        pltpu.sync_copy(o1_hbm.at[pl.ds(s * ROWS_PT, ROWS_PT)],
                        o1_sh.at[pl.ds(s * ROWS_PT, ROWS_PT)])

    @pl.when(s == NS - 1)
    def _():
        pltpu.sync_copy(o1_hbm.at[pl.ds((NS - 1) * ROWS_PT, lastn)],
                        o1_sh.at[pl.ds((NS - 1) * ROWS_PT, lastn)])

    _zero_acc_slice(rows, acc_sh, s)
    plsc.subcore_barrier()

    base = c * EHALF + s * ET
    pltpu.sync_copy(src_hbm.at[pl.ds(base, ET)], ebuf.at[pl.ds(0, ET)])
    pltpu.sync_copy(dst_hbm.at[pl.ds(base, ET)], ebuf.at[pl.ds(ET, ET)])

    def _fire(b, p):
        pltpu.async_copy(o1_sh.at[ebuf.at[pl.ds(b * PB, PB)]],
                         rows.at[p], gsem.at[p])

    _fire(0, 0)

    def _batch(b, _):
        p = lax.rem(b, 2)
        pn = 1 - p

        @pl.when(b + 1 < NB)
        def _():
            _fire(b + 1, pn)

        pltpu.make_async_copy(o1_sh.at[ebuf.at[pl.ds(b * PB, PB)]],
                              rows.at[p], gsem.at[p]).wait()
        for q in range(PB // L):
            sidx[q, pl.ds(0, L)] = ebuf[pl.ds(ET + b * PB + q * L, L)]
            pltpu.async_copy(rows.at[p, pl.ds(q * L, L)],
                             acc_sh.at[sidx.at[q]], wsem, add=True)
        for q in range(PB // L):
            pltpu.make_async_copy(rows.at[p, pl.ds(q * L, L)],
                                  acc_sh.at[sidx.at[q]], wsem).wait()
        return 0
    lax.fori_loop(0, NB, _batch, 0)
    plsc.subcore_barrier()

    pltpu.sync_copy(acc_sh.at[pl.ds(s * ROWS_PT, ROWS_PT)],
                    out_hbm.at[pl.ds(c * ACC_ROWS + s * ROWS_PT, ROWS_PT)])


_MESH = plsc.VectorSubcoreMesh(core_axis_name="c", subcore_axis_name="s",
                               num_cores=NC, num_subcores=NS)

_SC_PARAMS = pltpu.CompilerParams(needs_layout_passes=False,
                                  use_tc_tiling_on_sc=False)

_sc1 = functools.partial(
    pl.kernel, _sc1_body,
    out_type=jax.ShapeDtypeStruct((NC * ACC_ROWS, H), _f32),
    mesh=_MESH,
    compiler_params=_SC_PARAMS,
    scratch_types=[
        pltpu.VMEM((3 * ET,), _i32),
        pltpu.VMEM((ET,), _i32),
        pltpu.VMEM((ET,), _i32),
        pltpu.VMEM((PB // L, L), _i32),
        pltpu.VMEM((L,), _f32),
        pltpu.VMEM((2, PB), _f32),
        pltpu.VMEM((CNT_PT,), _f32),
        pltpu.VMEM((2, PB, H), _f32),
        pltpu.SemaphoreType.DMA((2,)),
        pltpu.SemaphoreType.DMA((2,)),
        pltpu.SemaphoreType.DMA,
        pltpu.VMEM_SHARED((CNTSZ,), _f32),
        pltpu.VMEM_SHARED((ACC_ROWS, H), _f32),
    ],
)

_sc2 = functools.partial(
    pl.kernel, _sc2_body,
    out_type=jax.ShapeDtypeStruct((NC * ACC_ROWS, H), _f32),
    mesh=_MESH,
    compiler_params=_SC_PARAMS,
    scratch_types=[
        pltpu.VMEM((2 * ET,), _i32),
        pltpu.VMEM((PB // L, L), _i32),
        pltpu.VMEM((2, PB, H), _f32),
        pltpu.SemaphoreType.DMA((2,)),
        pltpu.SemaphoreType.DMA,
        pltpu.VMEM_SHARED((ACC_ROWS, H), _f32),
        pltpu.VMEM_SHARED((ACC_ROWS, H), _f32),
    ],
)


def kernel(x, edge_index, edge_norm, edge_type, seq_lengths, umask,
           nodal_attn, avec, comp, basis, root, bias1, w_rel, w_root,
           bias2, w_lin, b_lin, w_fc, b_fc):
    del edge_norm, seq_lengths, umask, nodal_attn, avec

    src = edge_index[0].astype(_i32)
    dst = edge_index[1].astype(_i32)
    et = edge_type.astype(_i32)
    pad = EPAD - E
    srcp = jnp.concatenate([src, jnp.zeros((pad,), _i32)])
    dstp = jnp.concatenate([dst, jnp.full((pad,), N, _i32)])
    etp = jnp.concatenate([et, jnp.full((pad,), R, _i32)])

    basis2 = basis.reshape(basis.shape[0], D * H)
    w_flat = pl.pallas_call(
        _w_body,
        out_shape=jax.ShapeDtypeStruct((R, D * H), _f32),
    )(comp, basis2)
    w_all = w_flat.reshape(R, D, H)

    xw = pl.pallas_call(
        _xw_body,
        grid=(R,),
        in_specs=[
            pl.BlockSpec((N, D), lambda r: (0, 0)),
            pl.BlockSpec((1, D, H), lambda r: (r, 0, 0)),
        ],
        out_specs=pl.BlockSpec((1, N, H), lambda r: (r, 0, 0)),
        out_shape=jax.ShapeDtypeStruct((R, N, H), _f32),
    )(x, w_all)
    xw2d = xw.reshape(RN, H)

    relpart = _sc1()(etp, dstp, srcp, xw2d)

    out1 = pl.pallas_call(
        _out1_body,
        out_shape=jax.ShapeDtypeStruct((N, H), _f32),
    )(x, relpart[:N], relpart[ACC_ROWS:ACC_ROWS + N], root, bias1[None, :])

    aggpart = _sc2()(srcp, dstp, out1)

    log_prob = pl.pallas_call(
        _head_body,
        out_shape=jax.ShapeDtypeStruct((N, b_fc.shape[0]), _f32),
    )(x, out1, aggpart[:N], aggpart[ACC_ROWS:ACC_ROWS + N],
      w_rel, w_root, bias2[None, :], w_lin[:D], w_lin[D:],
      b_lin[None, :], w_fc, b_fc[None, :])
    return log_prob


# speedup vs baseline: 8.0337x; 1.2157x over previous
import functools

import jax
import jax.numpy as jnp
from jax import lax
from jax.experimental import pallas as pl
from jax.experimental.pallas import tpu as pltpu
from jax.experimental.pallas import tpu_sc as plsc

N = 10000
E = 320000
D = 128
H = 64
R = 16
RN = R * N

NC = 2
NS = 16
L = 16

PB = 128
SCW = 64
NSC = PB // SCW
NB = 79
ET = NB * PB
EPAD = NC * NS * ET
EHALF = NS * ET
NBA = 2 * NB

ACC_ROWS = 10112
ROWS_PT = ACC_ROWS // NS
CNT_PT = 10640
CNTSZ = NS * CNT_PT
CZ = CNT_PT // 5

_i32 = jnp.int32
_f32 = jnp.float32


def _w_body(comp_ref, basis_ref, w_ref):
    w_ref[...] = jnp.dot(comp_ref[...], basis_ref[...],
                         preferred_element_type=_f32)


def _xw_body(x_ref, w_ref, xw_ref):
    xw_ref[...] = jnp.dot(x_ref[...], w_ref[0],
                          preferred_element_type=_f32)[None]


def _out1_body(x_ref, p0_ref, p1_ref, root_ref, b1_ref, o_ref):
    o_ref[...] = (p0_ref[...] + p1_ref[...] + b1_ref[...]
                  + jnp.dot(x_ref[...], root_ref[...],
                            preferred_element_type=_f32))


def _head_body(x_ref, o1_ref, a0_ref, a1_ref, wrel_ref, wroot_ref, b2_ref,
               wlx_ref, wlh_ref, bl_ref, wfc_ref, bfc_ref, o_ref):
    agg2 = a0_ref[...] + a1_ref[...]
    out2 = (jnp.dot(agg2, wrel_ref[...], preferred_element_type=_f32)
            + jnp.dot(o1_ref[...], wroot_ref[...], preferred_element_type=_f32)
            + b2_ref[...])
    hidden = (jnp.dot(x_ref[...], wlx_ref[...], preferred_element_type=_f32)
              + jnp.dot(out2, wlh_ref[...], preferred_element_type=_f32)
              + bl_ref[...])
    hidden = jnp.maximum(hidden, 0.0)
    logits = jnp.dot(hidden, wfc_ref[...], preferred_element_type=_f32) \
        + bfc_ref[...]
    m = jnp.max(logits, axis=1, keepdims=True)
    s = logits - m
    lse = jnp.log(jnp.sum(jnp.exp(s), axis=1, keepdims=True))
    o_ref[...] = s - lse


def _zero_acc_slice(rows, acc_sh, s):
    def _fill_rows(i, _):
        for k in range(H // L):
            rows[0, i, pl.ds(k * L, L)] = jnp.zeros((L,), _f32)
        return 0
    lax.fori_loop(0, PB, _fill_rows, 0)
    for q in range(ROWS_PT // PB):
        pltpu.sync_copy(rows.at[0],
                        acc_sh.at[pl.ds(s * ROWS_PT + q * PB, PB)])
    rem = ROWS_PT % PB
    if rem:
        pltpu.sync_copy(
            rows.at[0, pl.ds(0, rem)],
            acc_sh.at[pl.ds(s * ROWS_PT + (ROWS_PT // PB) * PB, rem)])


def _sc1_body(et_hbm, dst_hbm, src_hbm, xw_hbm, out_hbm,
              ebuf, idx_c, idx_g, idx_c2, ones, sval, zc, rows,
              gsem, ssem, wsem, cnt_sh, acc_sh):
    c = lax.axis_index("c")
    s = lax.axis_index("s")

    lanes = lax.broadcasted_iota(_i32, (L,), 0)

    def _fill_zc(i, _):
        zc[pl.ds(i * L, L)] = jnp.zeros((L,), _f32)
        return 0
    lax.fori_loop(0, CZ // L, _fill_zc, 0)

    for r in range(SCW // L):
        ones[pl.ds(r * L, L)] = jnp.ones((L,), _f32)

    for u in range(CNT_PT // CZ):
        pltpu.sync_copy(zc, cnt_sh.at[pl.ds(s * CNT_PT + u * CZ, CZ)])
    _zero_acc_slice(rows, acc_sh, s)
    plsc.subcore_barrier()

    scope_a = jax.named_scope("sc1_pass_a")
    scope_a.__enter__()
    NSA = ET // SCW

    def _stage_a(st, _):
        abase = s * (2 * ET) + st * ET
        pltpu.sync_copy(et_hbm.at[pl.ds(abase, ET)], ebuf.at[pl.ds(0, ET)])
        pltpu.sync_copy(dst_hbm.at[pl.ds(abase, ET)], ebuf.at[pl.ds(ET, ET)])

        def _cidx(row, _):
            for r in range(SCW // L):
                o = row * SCW + r * L
                t16 = ebuf[pl.ds(o, L)]
                d16 = ebuf[pl.ds(ET + o, L)]
                idx_c2[row, pl.ds(r * L, L)] = t16 * N + d16
            return 0
        lax.fori_loop(0, NSA, _cidx, 0)

        def _scat(b, _):
            pltpu.async_copy(ones, cnt_sh.at[idx_c2.at[b]], wsem, add=True)
            return 0
        lax.fori_loop(0, NSA, _scat, 0)

        def _drain(b, _):
            pltpu.make_async_copy(ones, cnt_sh.at[idx_c2.at[b]],
                                  wsem).wait()
            return 0
        lax.fori_loop(0, NSA, _drain, 0)
        return 0
    lax.fori_loop(0, 2, _stage_a, 0)
    plsc.subcore_barrier()
    scope_a.__exit__(None, None, None)

    def _recip(i, _):
        v = zc[pl.ds(i * L, L)]
        zc[pl.ds(i * L, L)] = 1.0 / jnp.maximum(v, 1.0)
        return 0

    for u in range(CNT_PT // CZ):
        pltpu.sync_copy(cnt_sh.at[pl.ds(s * CNT_PT + u * CZ, CZ)], zc)
        lax.fori_loop(0, CZ // L, _recip, 0)
        pltpu.sync_copy(zc, cnt_sh.at[pl.ds(s * CNT_PT + u * CZ, CZ)])
    plsc.subcore_barrier()

    scope_b = jax.named_scope("sc1_pass_b")
    scope_b.__enter__()
    base = c * EHALF + s * ET
    pltpu.sync_copy(et_hbm.at[pl.ds(base, ET)], ebuf.at[pl.ds(0, ET)])
    pltpu.sync_copy(dst_hbm.at[pl.ds(base, ET)], ebuf.at[pl.ds(ET, ET)])
    pltpu.sync_copy(src_hbm.at[pl.ds(base, ET)], ebuf.at[pl.ds(2 * ET, ET)])

    def _bidx(row, _):
        for r in range(SCW // L):
            o = row * SCW + r * L
            t16 = ebuf[pl.ds(o, L)]
            d16 = ebuf[pl.ds(ET + o, L)]
            s16 = ebuf[pl.ds(2 * ET + o, L)]
            idx_c[pl.ds(o, L)] = t16 * N + d16
            idx_g[pl.ds(o, L)] = jnp.minimum(t16 * N + s16, RN - 1)
            idx_c2[row, pl.ds(r * L, L)] = d16
        return 0
    lax.fori_loop(0, NSA, _bidx, 0)

    fidx = [lanes + f * L for f in range(H // L)]

    def _fire(b, p):
        pltpu.async_copy(xw_hbm.at[idx_g.at[pl.ds(b * PB, PB)]],
                         rows.at[p], gsem.at[p])
        pltpu.async_copy(cnt_sh.at[idx_c.at[pl.ds(b * PB, PB)]],
                         sval.at[p], ssem.at[p])

    _fire(0, 0)

    def _batch(b, _):
        p = lax.rem(b, 2)
        pn = 1 - p

        @pl.when(b + 1 < NB)
        def _():
            _fire(b + 1, pn)

        pltpu.make_async_copy(xw_hbm.at[idx_g.at[pl.ds(b * PB, PB)]],
                              rows.at[p], gsem.at[p]).wait()
        pltpu.make_async_copy(cnt_sh.at[idx_c.at[pl.ds(b * PB, PB)]],
                              sval.at[p], ssem.at[p]).wait()
        def _grp(j, _):
            sv = sval[p, pl.ds(j * L, L)]
            for k in range(L):
                e = j * L + k
                ss = sv[k]
                for f in range(H // L):
                    v = rows[p, e, pl.ds(f * L, L)]
                    rows[p, e, pl.ds(f * L, L)] = v * ss
            return 0
        lax.fori_loop(0, PB // L, _grp, 0)

        for q in range(NSC):
            pltpu.async_copy(rows.at[p, pl.ds(q * SCW, SCW)],
                             acc_sh.at[idx_c2.at[b * NSC + q]], wsem,
                             add=True)
        for q in range(NSC):
            pltpu.make_async_copy(rows.at[p, pl.ds(q * SCW, SCW)],
                                  acc_sh.at[idx_c2.at[b * NSC + q]],
                                  wsem).wait()
        return 0
    lax.fori_loop(0, NB, _batch, 0)
    plsc.subcore_barrier()
    scope_b.__exit__(None, None, None)

    pltpu.sync_copy(acc_sh.at[pl.ds(s * ROWS_PT, ROWS_PT)],
                    out_hbm.at[pl.ds(c * ACC_ROWS + s * ROWS_PT, ROWS_PT)])


def _sc2_body(src_hbm, dst_hbm, o1_hbm, out_hbm,
              ebuf, sidx, rows, gsem, wsem, acc_sh, o1_sh):
    c = lax.axis_index("c")
    s = lax.axis_index("s")

    lastn = N - (NS - 1) * ROWS_PT

    @pl.when(s < NS - 1)
    def _():
        pltpu.sync_copy(o1_hbm.at[pl.ds(s * ROWS_PT, ROWS_PT)],
                        o1_sh.at[pl.ds(s * ROWS_PT, ROWS_PT)])

    @pl.when(s == NS - 1)
    def _():
        pltpu.sync_copy(o1_hbm.at[pl.ds((NS - 1) * ROWS_PT, lastn)],
                        o1_sh.at[pl.ds((NS - 1) * ROWS_PT, lastn)])

    _zero_acc_slice(rows, acc_sh, s)
    plsc.subcore_barrier()

    base = c * EHALF + s * ET
    pltpu.sync_copy(src_hbm.at[pl.ds(base, ET)], ebuf.at[pl.ds(0, ET)])
    pltpu.sync_copy(dst_hbm.at[pl.ds(base, ET)], ebuf.at[pl.ds(ET, ET)])

    def _fire(b, p):
        pltpu.async_copy(o1_sh.at[ebuf.at[pl.ds(b * PB, PB)]],
                         rows.at[p], gsem.at[p])

    _fire(0, 0)

    def _batch(b, _):
        p = lax.rem(b, 2)
        pn = 1 - p

        @pl.when(b + 1 < NB)
        def _():
            _fire(b + 1, pn)

        pltpu.make_async_copy(o1_sh.at[ebuf.at[pl.ds(b * PB, PB)]],
                              rows.at[p], gsem.at[p]).wait()
        for q in range(NSC):
            for r in range(SCW // L):
                sidx[q, pl.ds(r * L, L)] = \
                    ebuf[pl.ds(ET + b * PB + q * SCW + r * L, L)]
            pltpu.async_copy(rows.at[p, pl.ds(q * SCW, SCW)],
                             acc_sh.at[sidx.at[q]], wsem, add=True)
        for q in range(NSC):
            pltpu.make_async_copy(rows.at[p, pl.ds(q * SCW, SCW)],
                                  acc_sh.at[sidx.at[q]], wsem).wait()
        return 0
    lax.fori_loop(0, NB, _batch, 0)
    plsc.subcore_barrier()

    pltpu.sync_copy(acc_sh.at[pl.ds(s * ROWS_PT, ROWS_PT)],
                    out_hbm.at[pl.ds(c * ACC_ROWS + s * ROWS_PT, ROWS_PT)])


_MESH = plsc.VectorSubcoreMesh(core_axis_name="c", subcore_axis_name="s",
                               num_cores=NC, num_subcores=NS)

_SC_PARAMS = pltpu.CompilerParams(needs_layout_passes=False,
                                  use_tc_tiling_on_sc=False)

_sc1 = functools.partial(
    pl.kernel, _sc1_body,
    out_type=jax.ShapeDtypeStruct((NC * ACC_ROWS, H), _f32),
    mesh=_MESH,
    compiler_params=_SC_PARAMS,
    scratch_types=[
        pltpu.VMEM((3 * ET,), _i32),
        pltpu.VMEM((ET,), _i32),
        pltpu.VMEM((ET,), _i32),
        pltpu.VMEM((ET // SCW, SCW), _i32),
        pltpu.VMEM((SCW,), _f32),
        pltpu.VMEM((2, PB), _f32),
        pltpu.VMEM((CZ,), _f32),
        pltpu.VMEM((2, PB, H), _f32),
        pltpu.SemaphoreType.DMA((2,)),
        pltpu.SemaphoreType.DMA((2,)),
        pltpu.SemaphoreType.DMA,
        pltpu.VMEM_SHARED((CNTSZ,), _f32),
        pltpu.VMEM_SHARED((ACC_ROWS, H), _f32),
    ],
)

_sc2 = functools.partial(
    pl.kernel, _sc2_body,
    out_type=jax.ShapeDtypeStruct((NC * ACC_ROWS, H), _f32),
    mesh=_MESH,
    compiler_params=_SC_PARAMS,
    scratch_types=[
        pltpu.VMEM((2 * ET,), _i32),
        pltpu.VMEM((NSC, SCW), _i32),
        pltpu.VMEM((2, PB, H), _f32),
        pltpu.SemaphoreType.DMA((2,)),
        pltpu.SemaphoreType.DMA,
        pltpu.VMEM_SHARED((ACC_ROWS, H), _f32),
        pltpu.VMEM_SHARED((ACC_ROWS, H), _f32),
    ],
)


def kernel(x, edge_index, edge_norm, edge_type, seq_lengths, umask,
           nodal_attn, avec, comp, basis, root, bias1, w_rel, w_root,
           bias2, w_lin, b_lin, w_fc, b_fc):
    del edge_norm, seq_lengths, umask, nodal_attn, avec

    src = edge_index[0].astype(_i32)
    dst = edge_index[1].astype(_i32)
    et = edge_type.astype(_i32)
    pad = EPAD - E
    srcp = jnp.concatenate([src, jnp.zeros((pad,), _i32)])
    dstp = jnp.concatenate([dst, jnp.full((pad,), N, _i32)])
    etp = jnp.concatenate([et, jnp.full((pad,), R, _i32)])

    basis2 = basis.reshape(basis.shape[0], D * H)
    w_flat = pl.pallas_call(
        _w_body,
        out_shape=jax.ShapeDtypeStruct((R, D * H), _f32),
    )(comp, basis2)
    w_all = w_flat.reshape(R, D, H)

    xw = pl.pallas_call(
        _xw_body,
        grid=(R,),
        in_specs=[
            pl.BlockSpec((N, D), lambda r: (0, 0)),
            pl.BlockSpec((1, D, H), lambda r: (r, 0, 0)),
        ],
        out_specs=pl.BlockSpec((1, N, H), lambda r: (r, 0, 0)),
        out_shape=jax.ShapeDtypeStruct((R, N, H), _f32),
    )(x, w_all)
    xw2d = xw.reshape(RN, H)

    relpart = _sc1()(etp, dstp, srcp, xw2d)

    out1 = pl.pallas_call(
        _out1_body,
        out_shape=jax.ShapeDtypeStruct((N, H), _f32),
    )(x, relpart[:N], relpart[ACC_ROWS:ACC_ROWS + N], root, bias1[None, :])

    aggpart = _sc2()(srcp, dstp, out1)

    log_prob = pl.pallas_call(
        _head_body,
        out_shape=jax.ShapeDtypeStruct((N, b_fc.shape[0]), _f32),
    )(x, out1, aggpart[:N], aggpart[ACC_ROWS:ACC_ROWS + N],
      w_rel, w_root, bias2[None, :], w_lin[:D], w_lin[D:],
      b_lin[None, :], w_fc, b_fc[None, :])
    return log_prob
